# Initial kernel scaffold; baseline (speedup 1.0000x reference)
import jax
import jax.numpy as jnp
from jax.experimental import pallas as pl


def kernel(x, We1, be1, We2, be2, We3, be3, Wd1, bd1, Wd2, bd2, Wd3, bd3):
    raise NotImplementedError("write your pallas kernel here")


# speedup vs baseline: 398.8418x; 398.8418x over previous
import functools
import math

import jax
import jax.numpy as jnp
from jax.experimental import pallas as pl

_B = 16
_N = 2048
_D = 3
_C = 128
_M = 1000
_OUT_PAD = 3072


def _gelu(t):
    return 0.5 * t * (1.0 + jax.lax.erf(t * (1.0 / math.sqrt(2.0))))


def _encoder_body(x_ref, w1_ref, b1_ref, w2_ref, b2_ref, w3_ref, b3_ref,
                  z_ref):
    xb = x_ref[0]
    w1 = w1_ref[...]
    h = (xb[:, 0:1] * w1[0:1, :]
         + xb[:, 1:2] * w1[1:2, :]
         + xb[:, 2:3] * w1[2:3, :]
         + b1_ref[...])
    h = _gelu(h)
    h = _gelu(jnp.dot(h, w2_ref[...], preferred_element_type=jnp.float32)
              + b2_ref[...])
    h = _gelu(jnp.dot(h, w3_ref[...], preferred_element_type=jnp.float32)
              + b3_ref[...])
    z_ref[0, 0, :] = jnp.max(h, axis=0)


def _decoder_body(z_ref, w1_ref, b1_ref, w2_ref, b2_ref, w3_ref, b3_ref,
                  o_ref):
    d = _gelu(jnp.dot(z_ref[...], w1_ref[...],
                      preferred_element_type=jnp.float32) + b1_ref[...])
    d = _gelu(jnp.dot(d, w2_ref[...],
                      preferred_element_type=jnp.float32) + b2_ref[...])
    o_ref[...] = (jnp.dot(d, w3_ref[...], preferred_element_type=jnp.float32)
                  + b3_ref[...])


@functools.partial(jax.jit, static_argnames=())
def kernel(x, We1, be1, We2, be2, We3, be3, Wd1, bd1, Wd2, bd2, Wd3, bd3):
    full = lambda s: pl.BlockSpec(s, lambda b: (0,) * len(s))
    z3 = pl.pallas_call(
        _encoder_body,
        grid=(_B,),
        in_specs=[
            pl.BlockSpec((1, _N, _D), lambda b: (b, 0, 0)),
            full((_D, _C)), full((1, _C)),
            full((_C, _C)), full((1, _C)),
            full((_C, _C)), full((1, _C)),
        ],
        out_specs=pl.BlockSpec((1, 1, _C), lambda b: (b, 0, 0)),
        out_shape=jax.ShapeDtypeStruct((_B, 1, _C), jnp.float32),
    )(x, We1, be1.reshape(1, _C), We2, be2.reshape(1, _C),
      We3, be3.reshape(1, _C))
    z = z3.reshape(_B, _C)

    Wd3p = jnp.pad(Wd3, ((0, 0), (0, _OUT_PAD - _M * _D)))
    bd3p = jnp.pad(bd3, (0, _OUT_PAD - _M * _D)).reshape(1, _OUT_PAD)
    out = pl.pallas_call(
        _decoder_body,
        in_specs=[
            pl.BlockSpec((_B, _C), lambda: (0, 0)),
            pl.BlockSpec((_C, _C), lambda: (0, 0)),
            pl.BlockSpec((1, _C), lambda: (0, 0)),
            pl.BlockSpec((_C, _C), lambda: (0, 0)),
            pl.BlockSpec((1, _C), lambda: (0, 0)),
            pl.BlockSpec((_C, _OUT_PAD), lambda: (0, 0)),
            pl.BlockSpec((1, _OUT_PAD), lambda: (0, 0)),
        ],
        out_specs=pl.BlockSpec((_B, _OUT_PAD), lambda: (0, 0)),
        out_shape=jax.ShapeDtypeStruct((_B, _OUT_PAD), jnp.float32),
    )(z, Wd1, bd1.reshape(1, _C), Wd2, bd2.reshape(1, _C), Wd3p, bd3p)
    x_ = out[:, : _M * _D].reshape(_B, _M, _D)
    return (x_, z)


# speedup vs baseline: 425.7214x; 1.0674x over previous
import functools
import math

import jax
import jax.numpy as jnp
from jax.experimental import pallas as pl

_B = 16
_N = 2048
_D = 3
_C = 128
_M = 1000


def _gelu(t):
    return 0.5 * t * (1.0 + jax.lax.erf(t * (1.0 / math.sqrt(2.0))))


def _body(x_ref, we1_ref, be1_ref, we2_ref, be2_ref, we3_ref, be3_ref,
          wd1_ref, bd1_ref, wd2_ref, bd2_ref, wd3_ref, bd3_ref,
          z_ref, o_ref):
    b = pl.program_id(0)

    @pl.when(b < _B)
    def _encode():
        xb = x_ref[0]
        w1 = we1_ref[...]
        h = (xb[:, 0:1] * w1[0:1, :]
             + xb[:, 1:2] * w1[1:2, :]
             + xb[:, 2:3] * w1[2:3, :]
             + be1_ref[...])
        h = _gelu(h)
        h = _gelu(jnp.dot(h, we2_ref[...], preferred_element_type=jnp.float32)
                  + be2_ref[...])
        a = (jnp.dot(h, we3_ref[...], preferred_element_type=jnp.float32)
             + be3_ref[...])
        zrow = jnp.maximum(_gelu(jnp.max(a, axis=0)),
                           _gelu(jnp.min(a, axis=0)))
        z_ref[pl.ds(b, 1), :] = zrow.reshape(1, _C)

    @pl.when(b == _B)
    def _decode():
        d = _gelu(jnp.dot(z_ref[...], wd1_ref[...],
                          preferred_element_type=jnp.float32) + bd1_ref[...])
        d = _gelu(jnp.dot(d, wd2_ref[...],
                          preferred_element_type=jnp.float32) + bd2_ref[...])
        o_ref[...] = (jnp.dot(d, wd3_ref[...],
                              preferred_element_type=jnp.float32)
                      + bd3_ref[...])


@functools.partial(jax.jit, static_argnames=())
def kernel(x, We1, be1, We2, be2, We3, be3, Wd1, bd1, Wd2, bd2, Wd3, bd3):
    full = lambda s: pl.BlockSpec(s, lambda b: (0,) * len(s))
    z, out = pl.pallas_call(
        _body,
        grid=(_B + 1,),
        in_specs=[
            pl.BlockSpec((1, _N, _D), lambda b: (jnp.minimum(b, _B - 1), 0, 0)),
            full((_D, _C)), full((1, _C)),
            full((_C, _C)), full((1, _C)),
            full((_C, _C)), full((1, _C)),
            full((_C, _C)), full((1, _C)),
            full((_C, _C)), full((1, _C)),
            full((_C, _M * _D)), full((1, _M * _D)),
        ],
        out_specs=[
            pl.BlockSpec((_B, _C), lambda b: (0, 0)),
            pl.BlockSpec((_B, _M * _D), lambda b: (0, 0)),
        ],
        out_shape=[
            jax.ShapeDtypeStruct((_B, _C), jnp.float32),
            jax.ShapeDtypeStruct((_B, _M * _D), jnp.float32),
        ],
    )(x, We1, be1.reshape(1, _C), We2, be2.reshape(1, _C),
      We3, be3.reshape(1, _C), Wd1, bd1.reshape(1, _C),
      Wd2, bd2.reshape(1, _C), Wd3, bd3.reshape(1, _M * _D))
    return (out.reshape(_B, _M, _D), z)


# speedup vs baseline: 595.3810x; 1.3985x over previous
import functools
import math

import jax
import jax.numpy as jnp
from jax.experimental import pallas as pl

_B = 16
_N = 2048
_D = 3
_C = 128
_M = 1000


def _gelu(t):
    return 0.5 * t * (1.0 + jax.lax.erf(t * (1.0 / math.sqrt(2.0))))


_BPS = 4
_NSTEPS = _B // _BPS


def _body(x_ref, we1_ref, be1_ref, we2_ref, be2_ref, we3_ref, be3_ref,
          wd1_ref, bd1_ref, wd2_ref, bd2_ref, wd3_ref, bd3_ref,
          z_ref, o_ref):
    b = pl.program_id(0)

    @pl.when(b < _NSTEPS)
    def _encode():
        xb = x_ref[...].reshape(_BPS * _N, _D)
        h = (jnp.dot(xb, we1_ref[...], preferred_element_type=jnp.float32)
             + be1_ref[...])
        h = _gelu(h)
        h = _gelu(jnp.dot(h, we2_ref[...], preferred_element_type=jnp.float32)
                  + be2_ref[...])
        a = (jnp.dot(h, we3_ref[...], preferred_element_type=jnp.float32)
             + be3_ref[...])
        a3 = a.reshape(_BPS, _N, _C)
        zrow = jnp.maximum(_gelu(jnp.max(a3, axis=1)),
                           _gelu(jnp.min(a3, axis=1)))
        z_ref[pl.ds(b * _BPS, _BPS), :] = zrow

    @pl.when(b == _NSTEPS)
    def _decode():
        d = _gelu(jnp.dot(z_ref[...], wd1_ref[...],
                          preferred_element_type=jnp.float32) + bd1_ref[...])
        d = _gelu(jnp.dot(d, wd2_ref[...],
                          preferred_element_type=jnp.float32) + bd2_ref[...])
        o_ref[...] = (jnp.dot(d, wd3_ref[...],
                              preferred_element_type=jnp.float32)
                      + bd3_ref[...])


@functools.partial(jax.jit, static_argnames=())
def kernel(x, We1, be1, We2, be2, We3, be3, Wd1, bd1, Wd2, bd2, Wd3, bd3):
    full = lambda s: pl.BlockSpec(s, lambda b: (0,) * len(s))
    z, out = pl.pallas_call(
        _body,
        grid=(_NSTEPS + 1,),
        in_specs=[
            pl.BlockSpec((_BPS, _N, _D),
                         lambda b: (jnp.minimum(b, _NSTEPS - 1), 0, 0)),
            full((_D, _C)), full((1, _C)),
            full((_C, _C)), full((1, _C)),
            full((_C, _C)), full((1, _C)),
            full((_C, _C)), full((1, _C)),
            full((_C, _C)), full((1, _C)),
            full((_C, _M * _D)), full((1, _M * _D)),
        ],
        out_specs=[
            pl.BlockSpec((_B, _C), lambda b: (0, 0)),
            pl.BlockSpec((_B, _M * _D), lambda b: (0, 0)),
        ],
        out_shape=[
            jax.ShapeDtypeStruct((_B, _C), jnp.float32),
            jax.ShapeDtypeStruct((_B, _M * _D), jnp.float32),
        ],
    )(x, We1, be1.reshape(1, _C), We2, be2.reshape(1, _C),
      We3, be3.reshape(1, _C), Wd1, bd1.reshape(1, _C),
      Wd2, bd2.reshape(1, _C), Wd3, bd3.reshape(1, _M * _D))
    return (out.reshape(_B, _M, _D), z)
